# baseline (speedup 1.0000x reference)
import jax
import jax.numpy as jnp
from jax.experimental import pallas as pl
from jax.experimental.pallas import tpu as pltpu

_EPS = 1e-16
_LANES = 256
_SUB_BLOCK = 512


def _gious_body(g_ref, q_ref, o_ref):
    g = [g_ref[c] for c in range(6)]
    q = [q_ref[c] for c in range(6)]

    glo = [g[c] - 0.5 * g[c + 3] for c in range(3)]
    ghi = [g[c] + 0.5 * g[c + 3] for c in range(3)]
    qlo = [q[c] - 0.5 * q[c + 3] for c in range(3)]
    qhi = [q[c] + 0.5 * q[c + 3] for c in range(3)]

    def iw(c):
        return jnp.minimum(glo[c], qhi[c]) - jnp.maximum(glo[c], qhi[c]) + _EPS

    def ih(c):
        return jnp.minimum(ghi[c], qhi[c]) - jnp.maximum(glo[c], qlo[c]) + _EPS

    def mw(c):
        return jnp.maximum(ghi[c], qhi[c]) - jnp.minimum(glo[c], qlo[c]) + _EPS

    iw0, iw1 = iw(0), iw(1)
    ih1, ih2 = ih(1), ih(2)
    p_iw0, p_iw1 = iw0 > 0.0, iw1 > 0.0
    p_ih1, p_ih2 = ih1 > 0.0, ih2 > 0.0

    inter_xoz = jnp.where(p_iw0 & p_ih2, iw0 * ih2, 0.0)
    inter_xoy = jnp.where(p_iw0 & p_ih1, iw0 * ih1, 0.0)
    inter_yoz = jnp.where(p_iw1 & p_ih2, iw1 * ih2, 0.0)

    mbr_xoz = mw(0) * mw(2)
    mbr_yoz = mw(1) * mw(2)

    union_xoz = g[3] * g[5] + q[3] * q[5] - inter_xoz
    union_yoz = g[4] * g[5] + q[4] * q[5] - inter_xoy

    gious_xoz = inter_xoz / union_xoz - (mbr_xoz - union_xoz) / mbr_xoz
    gious_yoz = inter_yoz / union_yoz - (mbr_yoz - union_yoz) / mbr_yoz

    mins = jnp.minimum(
        jnp.minimum(jnp.minimum(g[3], g[4]), jnp.minimum(g[5], q[3])),
        jnp.minimum(q[4], q[5]),
    )
    gious = (gious_xoz + 2.0 * gious_yoz) * (1.0 / 3.0)
    o_ref[...] = jnp.where(mins > 0.0, gious, 0.0)


def _gious_planes(gp, qp, sub_block):
    s = gp.shape[1]
    grid = (pl.cdiv(s, sub_block),)
    return pl.pallas_call(
        _gious_body,
        grid=grid,
        in_specs=[
            pl.BlockSpec((6, sub_block, gp.shape[2]), lambda i: (0, i, 0)),
            pl.BlockSpec((6, sub_block, gp.shape[2]), lambda i: (0, i, 0)),
        ],
        out_specs=pl.BlockSpec((sub_block, gp.shape[2]), lambda i: (i, 0)),
        out_shape=jax.ShapeDtypeStruct((s, gp.shape[2]), jnp.float32),
        compiler_params=pltpu.CompilerParams(
            dimension_semantics=("parallel",),
        ),
    )(gp, qp)


def kernel(gboxes, qboxes):
    n = gboxes.shape[0]
    lanes = _LANES
    pad = (-n) % lanes
    if pad:
        gboxes = jnp.pad(gboxes, ((0, pad), (0, 0)))
        qboxes = jnp.pad(qboxes, ((0, pad), (0, 0)))
    s = (n + pad) // lanes
    gp = gboxes[:, :6].T.reshape(6, s, lanes)
    qp = qboxes[:, :6].T.reshape(6, s, lanes)
    out = _gious_planes(gp, qp, _SUB_BLOCK)
    return out.reshape(-1, 1)[:n]


# speedup vs baseline: 11.8517x; 11.8517x over previous
import jax
import jax.numpy as jnp
from jax.experimental import pallas as pl
from jax.experimental.pallas import tpu as pltpu

_EPS = 1e-16
_BM = 1250


def _issue_plane_copies(g_hbm, q_hbm, buf, sems, slot, blk, bm):
    base = blk * bm
    for c in range(6):
        pltpu.make_async_copy(
            g_hbm.at[pl.ds(base, bm), c, :], buf.at[slot, c], sems.at[slot, c]
        ).start()
        pltpu.make_async_copy(
            q_hbm.at[pl.ds(base, bm), c, :], buf.at[slot, 6 + c], sems.at[slot, 6 + c]
        ).start()


def _gious_body(g_hbm, q_hbm, o_ref, buf, sems):
    i = pl.program_id(0)
    nblk = pl.num_programs(0)
    bm = buf.shape[2]
    slot = jax.lax.rem(i, 2)

    @pl.when(i == 0)
    def _():
        _issue_plane_copies(g_hbm, q_hbm, buf, sems, 0, i, bm)

    @pl.when(i + 1 < nblk)
    def _():
        _issue_plane_copies(g_hbm, q_hbm, buf, sems, 1 - slot, i + 1, bm)

    for c in range(12):
        pltpu.make_async_copy(
            buf.at[slot, c], buf.at[slot, c], sems.at[slot, c]
        ).wait()

    g = [buf[slot, c] for c in range(6)]
    q = [buf[slot, 6 + c] for c in range(6)]

    glo = [g[c] - 0.5 * g[c + 3] for c in range(3)]
    ghi = [g[c] + 0.5 * g[c + 3] for c in range(3)]
    qlo = [q[c] - 0.5 * q[c + 3] for c in range(3)]
    qhi = [q[c] + 0.5 * q[c + 3] for c in range(3)]

    def iw(c):
        return jnp.minimum(glo[c], qhi[c]) - jnp.maximum(glo[c], qhi[c]) + _EPS

    def ih(c):
        return jnp.minimum(ghi[c], qhi[c]) - jnp.maximum(glo[c], qlo[c]) + _EPS

    def mw(c):
        return jnp.maximum(ghi[c], qhi[c]) - jnp.minimum(glo[c], qlo[c]) + _EPS

    iw0, iw1 = iw(0), iw(1)
    ih1, ih2 = ih(1), ih(2)
    p_iw0, p_iw1 = iw0 > 0.0, iw1 > 0.0
    p_ih1, p_ih2 = ih1 > 0.0, ih2 > 0.0

    inter_xoz = jnp.where(p_iw0 & p_ih2, iw0 * ih2, 0.0)
    inter_xoy = jnp.where(p_iw0 & p_ih1, iw0 * ih1, 0.0)
    inter_yoz = jnp.where(p_iw1 & p_ih2, iw1 * ih2, 0.0)

    mbr_xoz = mw(0) * mw(2)
    mbr_yoz = mw(1) * mw(2)

    union_xoz = g[3] * g[5] + q[3] * q[5] - inter_xoz
    union_yoz = g[4] * g[5] + q[4] * q[5] - inter_xoy

    gious_xoz = inter_xoz / union_xoz - (mbr_xoz - union_xoz) / mbr_xoz
    gious_yoz = inter_yoz / union_yoz - (mbr_yoz - union_yoz) / mbr_yoz

    mins = jnp.minimum(
        jnp.minimum(jnp.minimum(g[3], g[4]), jnp.minimum(g[5], q[3])),
        jnp.minimum(q[4], q[5]),
    )
    gious = (gious_xoz + 2.0 * gious_yoz) * (1.0 / 3.0)
    o_ref[...] = jnp.where(mins > 0.0, gious, 0.0).reshape(o_ref.shape)


def kernel(gboxes, qboxes):
    n = gboxes.shape[0]
    m = n // 128
    bm = _BM if m % _BM == 0 else m
    gv = gboxes.reshape(m, 128, 7).transpose(0, 2, 1)
    qv = qboxes.reshape(m, 128, 7).transpose(0, 2, 1)
    out = pl.pallas_call(
        _gious_body,
        grid=(m // bm,),
        in_specs=[
            pl.BlockSpec(memory_space=pl.ANY),
            pl.BlockSpec(memory_space=pl.ANY),
        ],
        out_specs=pl.BlockSpec((bm, 1, 128), lambda i: (i, 0, 0)),
        out_shape=jax.ShapeDtypeStruct((m, 1, 128), jnp.float32),
        scratch_shapes=[
            pltpu.VMEM((2, 12, bm, 128), jnp.float32),
            pltpu.SemaphoreType.DMA((2, 12)),
        ],
        compiler_params=pltpu.CompilerParams(
            dimension_semantics=("arbitrary",),
        ),
    )(gv, qv)
    return out.reshape(n, 1)


# speedup vs baseline: 12.4543x; 1.0508x over previous
import jax
import jax.numpy as jnp
from jax.experimental import pallas as pl
from jax.experimental.pallas import tpu as pltpu

_EPS = 1e-16
_BM = 1250


def _issue_plane_copies(g_hbm, q_hbm, buf, sems, slot, blk, bm):
    base = blk * bm
    for c in range(6):
        pltpu.make_async_copy(
            g_hbm.at[pl.ds(base, bm), c, :], buf.at[slot, c], sems.at[slot, c]
        ).start()
        pltpu.make_async_copy(
            q_hbm.at[pl.ds(base, bm), c, :], buf.at[slot, 6 + c], sems.at[slot, 6 + c]
        ).start()


def _gious_body(g_hbm, q_hbm, o_hbm, buf, sems, obuf, osems):
    i = pl.program_id(0)
    nblk = pl.num_programs(0)
    bm = buf.shape[2]
    slot = jax.lax.rem(i, 2)

    @pl.when(i == 0)
    def _():
        _issue_plane_copies(g_hbm, q_hbm, buf, sems, 0, i, bm)

    @pl.when(i + 1 < nblk)
    def _():
        _issue_plane_copies(g_hbm, q_hbm, buf, sems, 1 - slot, i + 1, bm)

    @pl.when(i >= 2)
    def _():
        pltpu.make_async_copy(
            obuf.at[slot], obuf.at[slot], osems.at[slot]
        ).wait()

    for c in range(12):
        pltpu.make_async_copy(
            buf.at[slot, c], buf.at[slot, c], sems.at[slot, c]
        ).wait()

    g = [buf[slot, c] for c in range(6)]
    q = [buf[slot, 6 + c] for c in range(6)]

    glo = [g[c] - 0.5 * g[c + 3] for c in range(3)]
    ghi = [g[c] + 0.5 * g[c + 3] for c in range(3)]
    qlo = [q[c] - 0.5 * q[c + 3] for c in range(3)]
    qhi = [q[c] + 0.5 * q[c + 3] for c in range(3)]

    def iw(c):
        return jnp.minimum(glo[c], qhi[c]) - jnp.maximum(glo[c], qhi[c]) + _EPS

    def ih(c):
        return jnp.minimum(ghi[c], qhi[c]) - jnp.maximum(glo[c], qlo[c]) + _EPS

    def mw(c):
        return jnp.maximum(ghi[c], qhi[c]) - jnp.minimum(glo[c], qlo[c]) + _EPS

    iw0, iw1 = iw(0), iw(1)
    ih1, ih2 = ih(1), ih(2)
    p_iw0, p_iw1 = iw0 > 0.0, iw1 > 0.0
    p_ih1, p_ih2 = ih1 > 0.0, ih2 > 0.0

    inter_xoz = jnp.where(p_iw0 & p_ih2, iw0 * ih2, 0.0)
    inter_xoy = jnp.where(p_iw0 & p_ih1, iw0 * ih1, 0.0)
    inter_yoz = jnp.where(p_iw1 & p_ih2, iw1 * ih2, 0.0)

    mbr_xoz = mw(0) * mw(2)
    mbr_yoz = mw(1) * mw(2)

    union_xoz = g[3] * g[5] + q[3] * q[5] - inter_xoz
    union_yoz = g[4] * g[5] + q[4] * q[5] - inter_xoy

    gious_xoz = inter_xoz / union_xoz - (mbr_xoz - union_xoz) / mbr_xoz
    gious_yoz = inter_yoz / union_yoz - (mbr_yoz - union_yoz) / mbr_yoz

    mins = jnp.minimum(
        jnp.minimum(jnp.minimum(g[3], g[4]), jnp.minimum(g[5], q[3])),
        jnp.minimum(q[4], q[5]),
    )
    gious = (gious_xoz + 2.0 * gious_yoz) * (1.0 / 3.0)

    obuf[slot] = jnp.where(mins > 0.0, gious, 0.0)
    out_cp = pltpu.make_async_copy(
        obuf.at[slot], o_hbm.at[pl.ds(i * bm, bm), 0, :], osems.at[slot]
    )
    out_cp.start()

    @pl.when(i == nblk - 1)
    def _():
        out_cp.wait()

        @pl.when(nblk > 1)
        def _():
            pltpu.make_async_copy(
                obuf.at[1 - slot], obuf.at[1 - slot], osems.at[1 - slot]
            ).wait()


def kernel(gboxes, qboxes):
    n = gboxes.shape[0]
    m = n // 128
    bm = _BM if m % _BM == 0 else m
    gv = gboxes.reshape(m, 128, 7).transpose(0, 2, 1)
    qv = qboxes.reshape(m, 128, 7).transpose(0, 2, 1)
    out = pl.pallas_call(
        _gious_body,
        grid=(m // bm,),
        in_specs=[
            pl.BlockSpec(memory_space=pl.ANY),
            pl.BlockSpec(memory_space=pl.ANY),
        ],
        out_specs=pl.BlockSpec(memory_space=pl.ANY),
        out_shape=jax.ShapeDtypeStruct((m, 1, 128), jnp.float32),
        scratch_shapes=[
            pltpu.VMEM((2, 12, bm, 128), jnp.float32),
            pltpu.SemaphoreType.DMA((2, 12)),
            pltpu.VMEM((2, bm, 128), jnp.float32),
            pltpu.SemaphoreType.DMA((2,)),
        ],
        compiler_params=pltpu.CompilerParams(
            dimension_semantics=("arbitrary",),
        ),
    )(gv, qv)
    return out.reshape(n, 1)


# speedup vs baseline: 13.8141x; 1.1092x over previous
import functools

import jax
import jax.numpy as jnp
from jax.experimental import pallas as pl
from jax.experimental.pallas import tpu as pltpu

_EPS = 1e-16
_BM = 1250


def _issue_plane_copies(g_hbm, q_hbm, buf, sems, slot, blk, bm):
    base = blk * bm
    for c in range(6):
        pltpu.make_async_copy(
            g_hbm.at[pl.ds(base, bm), c, :], buf.at[slot, c], sems.at[slot, c]
        ).start()
        pltpu.make_async_copy(
            q_hbm.at[pl.ds(base, bm), c, :], buf.at[slot, 6 + c], sems.at[slot, 6 + c]
        ).start()


def _gious_body(g_hbm, q_hbm, o_hbm, buf, sems, obuf, osems, *, nblk):
    i = pl.program_id(0)
    bm = buf.shape[2]
    nslot = buf.shape[0]
    slot = jax.lax.rem(i, nslot)

    @pl.when(i == 0)
    def _():
        for k in range(min(nslot - 1, nblk)):
            _issue_plane_copies(g_hbm, q_hbm, buf, sems, k, k, bm)

    @pl.when(i + nslot - 1 < nblk)
    def _():
        nxt = jax.lax.rem(i + nslot - 1, nslot)
        _issue_plane_copies(g_hbm, q_hbm, buf, sems, nxt, i + nslot - 1, bm)

    @pl.when(i >= nslot)
    def _():
        pltpu.make_async_copy(
            obuf.at[slot], obuf.at[slot], osems.at[slot]
        ).wait()

    for c in range(12):
        pltpu.make_async_copy(
            buf.at[slot, c], buf.at[slot, c], sems.at[slot, c]
        ).wait()

    g = [buf[slot, c] for c in range(6)]
    q = [buf[slot, 6 + c] for c in range(6)]

    glo = [g[c] - 0.5 * g[c + 3] for c in range(3)]
    ghi = [g[c] + 0.5 * g[c + 3] for c in range(3)]
    qlo = [q[c] - 0.5 * q[c + 3] for c in range(3)]
    qhi = [q[c] + 0.5 * q[c + 3] for c in range(3)]

    def iw(c):
        return jnp.minimum(glo[c], qhi[c]) - jnp.maximum(glo[c], qhi[c]) + _EPS

    def ih(c):
        return jnp.minimum(ghi[c], qhi[c]) - jnp.maximum(glo[c], qlo[c]) + _EPS

    def mw(c):
        return jnp.maximum(ghi[c], qhi[c]) - jnp.minimum(glo[c], qlo[c]) + _EPS

    iw0, iw1 = iw(0), iw(1)
    ih1, ih2 = ih(1), ih(2)
    p_iw0, p_iw1 = iw0 > 0.0, iw1 > 0.0
    p_ih1, p_ih2 = ih1 > 0.0, ih2 > 0.0

    inter_xoz = jnp.where(p_iw0 & p_ih2, iw0 * ih2, 0.0)
    inter_xoy = jnp.where(p_iw0 & p_ih1, iw0 * ih1, 0.0)
    inter_yoz = jnp.where(p_iw1 & p_ih2, iw1 * ih2, 0.0)

    mbr_xoz = mw(0) * mw(2)
    mbr_yoz = mw(1) * mw(2)

    union_xoz = g[3] * g[5] + q[3] * q[5] - inter_xoz
    union_yoz = g[4] * g[5] + q[4] * q[5] - inter_xoy

    gious_xoz = inter_xoz / union_xoz - (mbr_xoz - union_xoz) / mbr_xoz
    gious_yoz = inter_yoz / union_yoz - (mbr_yoz - union_yoz) / mbr_yoz

    mins = jnp.minimum(
        jnp.minimum(jnp.minimum(g[3], g[4]), jnp.minimum(g[5], q[3])),
        jnp.minimum(q[4], q[5]),
    )
    gious = (gious_xoz + 2.0 * gious_yoz) * (1.0 / 3.0)

    obuf[slot] = jnp.where(mins > 0.0, gious, 0.0)
    out_cp = pltpu.make_async_copy(
        obuf.at[slot], o_hbm.at[pl.ds(i * bm, bm), 0, :], osems.at[slot]
    )
    out_cp.start()

    @pl.when(i == nblk - 1)
    def _():
        out_cp.wait()
        for k in range(1, nslot):
            @pl.when(i >= k)
            def _(k=k):
                prev = jax.lax.rem(slot - k + nslot, nslot)
                pltpu.make_async_copy(
                    obuf.at[prev], obuf.at[prev], osems.at[prev]
                ).wait()


def kernel(gboxes, qboxes):
    n = gboxes.shape[0]
    m = n // 128
    bm = _BM if m % _BM == 0 else m
    gv = gboxes.reshape(m, 128, 7).transpose(0, 2, 1)
    qv = qboxes.reshape(m, 128, 7).transpose(0, 2, 1)
    nblk = m // bm
    out = pl.pallas_call(
        functools.partial(_gious_body, nblk=nblk),
        grid=(nblk,),
        in_specs=[
            pl.BlockSpec(memory_space=pl.ANY),
            pl.BlockSpec(memory_space=pl.ANY),
        ],
        out_specs=pl.BlockSpec(memory_space=pl.ANY),
        out_shape=jax.ShapeDtypeStruct((m, 1, 128), jnp.float32),
        scratch_shapes=[
            pltpu.VMEM((3, 12, bm, 128), jnp.float32),
            pltpu.SemaphoreType.DMA((3, 12)),
            pltpu.VMEM((3, bm, 128), jnp.float32),
            pltpu.SemaphoreType.DMA((3,)),
        ],
        compiler_params=pltpu.CompilerParams(
            dimension_semantics=("arbitrary",),
        ),
    )(gv, qv)
    return out.reshape(n, 1)


# speedup vs baseline: 13.9775x; 1.0118x over previous
import functools

import jax
import jax.numpy as jnp
from jax.experimental import pallas as pl
from jax.experimental.pallas import tpu as pltpu

_EPS = 1e-16
_BM = 625


def _issue_plane_copies(g_hbm, q_hbm, buf, sems, slot, blk, bm):
    base = blk * bm
    for c in range(6):
        pltpu.make_async_copy(
            g_hbm.at[pl.ds(base, bm), c, :], buf.at[slot, c], sems.at[slot, c]
        ).start()
        pltpu.make_async_copy(
            q_hbm.at[pl.ds(base, bm), c, :], buf.at[slot, 6 + c], sems.at[slot, 6 + c]
        ).start()


def _gious_body(g_hbm, q_hbm, o_hbm, buf, sems, obuf, osems, *, nblk):
    i = pl.program_id(0)
    bm = buf.shape[2]
    nslot = buf.shape[0]
    slot = jax.lax.rem(i, nslot)

    @pl.when(i == 0)
    def _():
        for k in range(min(nslot - 1, nblk)):
            _issue_plane_copies(g_hbm, q_hbm, buf, sems, k, k, bm)

    @pl.when(i + nslot - 1 < nblk)
    def _():
        nxt = jax.lax.rem(i + nslot - 1, nslot)
        _issue_plane_copies(g_hbm, q_hbm, buf, sems, nxt, i + nslot - 1, bm)

    @pl.when(i >= nslot)
    def _():
        pltpu.make_async_copy(
            obuf.at[slot], obuf.at[slot], osems.at[slot]
        ).wait()

    for c in range(12):
        pltpu.make_async_copy(
            buf.at[slot, c], buf.at[slot, c], sems.at[slot, c]
        ).wait()

    g = [buf[slot, c] for c in range(6)]
    q = [buf[slot, 6 + c] for c in range(6)]

    glo = [g[c] - 0.5 * g[c + 3] for c in range(3)]
    ghi = [g[c] + 0.5 * g[c + 3] for c in range(3)]
    qlo = [q[c] - 0.5 * q[c + 3] for c in range(3)]
    qhi = [q[c] + 0.5 * q[c + 3] for c in range(3)]

    def iw(c):
        return jnp.minimum(glo[c], qhi[c]) - jnp.maximum(glo[c], qhi[c]) + _EPS

    def ih(c):
        return jnp.minimum(ghi[c], qhi[c]) - jnp.maximum(glo[c], qlo[c]) + _EPS

    def mw(c):
        return jnp.maximum(ghi[c], qhi[c]) - jnp.minimum(glo[c], qlo[c]) + _EPS

    iw0, iw1 = iw(0), iw(1)
    ih1, ih2 = ih(1), ih(2)
    p_iw0, p_iw1 = iw0 > 0.0, iw1 > 0.0
    p_ih1, p_ih2 = ih1 > 0.0, ih2 > 0.0

    inter_xoz = jnp.where(p_iw0 & p_ih2, iw0 * ih2, 0.0)
    inter_xoy = jnp.where(p_iw0 & p_ih1, iw0 * ih1, 0.0)
    inter_yoz = jnp.where(p_iw1 & p_ih2, iw1 * ih2, 0.0)

    mbr_xoz = mw(0) * mw(2)
    mbr_yoz = mw(1) * mw(2)

    union_xoz = g[3] * g[5] + q[3] * q[5] - inter_xoz
    union_yoz = g[4] * g[5] + q[4] * q[5] - inter_xoy

    gious_xoz = inter_xoz / union_xoz - (mbr_xoz - union_xoz) / mbr_xoz
    gious_yoz = inter_yoz / union_yoz - (mbr_yoz - union_yoz) / mbr_yoz

    mins = jnp.minimum(
        jnp.minimum(jnp.minimum(g[3], g[4]), jnp.minimum(g[5], q[3])),
        jnp.minimum(q[4], q[5]),
    )
    gious = (gious_xoz + 2.0 * gious_yoz) * (1.0 / 3.0)

    obuf[slot] = jnp.where(mins > 0.0, gious, 0.0)
    out_cp = pltpu.make_async_copy(
        obuf.at[slot], o_hbm.at[pl.ds(i * bm, bm), 0, :], osems.at[slot]
    )
    out_cp.start()

    @pl.when(i == nblk - 1)
    def _():
        out_cp.wait()
        for k in range(1, nslot):
            @pl.when(i >= k)
            def _(k=k):
                prev = jax.lax.rem(slot - k + nslot, nslot)
                pltpu.make_async_copy(
                    obuf.at[prev], obuf.at[prev], osems.at[prev]
                ).wait()


def kernel(gboxes, qboxes):
    n = gboxes.shape[0]
    m = n // 128
    bm = _BM if m % _BM == 0 else m
    gv = gboxes.reshape(m, 128, 7).transpose(0, 2, 1)
    qv = qboxes.reshape(m, 128, 7).transpose(0, 2, 1)
    nblk = m // bm
    out = pl.pallas_call(
        functools.partial(_gious_body, nblk=nblk),
        grid=(nblk,),
        in_specs=[
            pl.BlockSpec(memory_space=pl.ANY),
            pl.BlockSpec(memory_space=pl.ANY),
        ],
        out_specs=pl.BlockSpec(memory_space=pl.ANY),
        out_shape=jax.ShapeDtypeStruct((m, 1, 128), jnp.float32),
        scratch_shapes=[
            pltpu.VMEM((4, 12, bm, 128), jnp.float32),
            pltpu.SemaphoreType.DMA((4, 12)),
            pltpu.VMEM((4, bm, 128), jnp.float32),
            pltpu.SemaphoreType.DMA((4,)),
        ],
        compiler_params=pltpu.CompilerParams(
            dimension_semantics=("arbitrary",),
        ),
    )(gv, qv)
    return out.reshape(n, 1)


# speedup vs baseline: 14.0470x; 1.0050x over previous
import functools

import jax
import jax.numpy as jnp
from jax.experimental import pallas as pl
from jax.experimental.pallas import tpu as pltpu

_EPS = 1e-16
_BM = 250


def _issue_plane_copies(g_hbm, q_hbm, buf, sems, slot, blk, bm):
    base = blk * bm
    for c in range(6):
        pltpu.make_async_copy(
            g_hbm.at[pl.ds(base, bm), c, :], buf.at[slot, c], sems.at[slot, c]
        ).start()
        pltpu.make_async_copy(
            q_hbm.at[pl.ds(base, bm), c, :], buf.at[slot, 6 + c], sems.at[slot, 6 + c]
        ).start()


def _gious_body(g_hbm, q_hbm, o_hbm, buf, sems, obuf, osems, *, nblk):
    i = pl.program_id(0)
    bm = buf.shape[2]
    nslot = buf.shape[0]
    slot = jax.lax.rem(i, nslot)

    @pl.when(i == 0)
    def _():
        for k in range(min(nslot - 1, nblk)):
            _issue_plane_copies(g_hbm, q_hbm, buf, sems, k, k, bm)

    @pl.when(i + nslot - 1 < nblk)
    def _():
        nxt = jax.lax.rem(i + nslot - 1, nslot)
        _issue_plane_copies(g_hbm, q_hbm, buf, sems, nxt, i + nslot - 1, bm)

    @pl.when(i >= nslot)
    def _():
        pltpu.make_async_copy(
            obuf.at[slot], obuf.at[slot], osems.at[slot]
        ).wait()

    for c in range(12):
        pltpu.make_async_copy(
            buf.at[slot, c], buf.at[slot, c], sems.at[slot, c]
        ).wait()

    g = [buf[slot, c] for c in range(6)]
    q = [buf[slot, 6 + c] for c in range(6)]

    glo = [g[c] - 0.5 * g[c + 3] for c in range(3)]
    ghi = [g[c] + 0.5 * g[c + 3] for c in range(3)]
    qlo = [q[c] - 0.5 * q[c + 3] for c in range(3)]
    qhi = [q[c] + 0.5 * q[c + 3] for c in range(3)]

    def iw(c):
        return jnp.minimum(glo[c], qhi[c]) - jnp.maximum(glo[c], qhi[c]) + _EPS

    def ih(c):
        return jnp.minimum(ghi[c], qhi[c]) - jnp.maximum(glo[c], qlo[c]) + _EPS

    def mw(c):
        return jnp.maximum(ghi[c], qhi[c]) - jnp.minimum(glo[c], qlo[c]) + _EPS

    iw0, iw1 = iw(0), iw(1)
    ih1, ih2 = ih(1), ih(2)
    p_iw0, p_iw1 = iw0 > 0.0, iw1 > 0.0
    p_ih1, p_ih2 = ih1 > 0.0, ih2 > 0.0

    inter_xoz = jnp.where(p_iw0 & p_ih2, iw0 * ih2, 0.0)
    inter_xoy = jnp.where(p_iw0 & p_ih1, iw0 * ih1, 0.0)
    inter_yoz = jnp.where(p_iw1 & p_ih2, iw1 * ih2, 0.0)

    mbr_xoz = mw(0) * mw(2)
    mbr_yoz = mw(1) * mw(2)

    union_xoz = g[3] * g[5] + q[3] * q[5] - inter_xoz
    union_yoz = g[4] * g[5] + q[4] * q[5] - inter_xoy

    gious_xoz = inter_xoz / union_xoz - (mbr_xoz - union_xoz) / mbr_xoz
    gious_yoz = inter_yoz / union_yoz - (mbr_yoz - union_yoz) / mbr_yoz

    mins = jnp.minimum(
        jnp.minimum(jnp.minimum(g[3], g[4]), jnp.minimum(g[5], q[3])),
        jnp.minimum(q[4], q[5]),
    )
    gious = (gious_xoz + 2.0 * gious_yoz) * (1.0 / 3.0)

    obuf[slot] = jnp.where(mins > 0.0, gious, 0.0)
    out_cp = pltpu.make_async_copy(
        obuf.at[slot], o_hbm.at[pl.ds(i * bm, bm), 0, :], osems.at[slot]
    )
    out_cp.start()

    @pl.when(i == nblk - 1)
    def _():
        out_cp.wait()
        for k in range(1, nslot):
            @pl.when(i >= k)
            def _(k=k):
                prev = jax.lax.rem(slot - k + nslot, nslot)
                pltpu.make_async_copy(
                    obuf.at[prev], obuf.at[prev], osems.at[prev]
                ).wait()


def kernel(gboxes, qboxes):
    n = gboxes.shape[0]
    m = n // 128
    bm = _BM if m % _BM == 0 else m
    gv = gboxes.reshape(m, 128, 7).transpose(0, 2, 1)
    qv = qboxes.reshape(m, 128, 7).transpose(0, 2, 1)
    nblk = m // bm
    out = pl.pallas_call(
        functools.partial(_gious_body, nblk=nblk),
        grid=(nblk,),
        in_specs=[
            pl.BlockSpec(memory_space=pl.ANY),
            pl.BlockSpec(memory_space=pl.ANY),
        ],
        out_specs=pl.BlockSpec(memory_space=pl.ANY),
        out_shape=jax.ShapeDtypeStruct((m, 1, 128), jnp.float32),
        scratch_shapes=[
            pltpu.VMEM((6, 12, bm, 128), jnp.float32),
            pltpu.SemaphoreType.DMA((6, 12)),
            pltpu.VMEM((6, bm, 128), jnp.float32),
            pltpu.SemaphoreType.DMA((6,)),
        ],
        compiler_params=pltpu.CompilerParams(
            dimension_semantics=("arbitrary",),
        ),
    )(gv, qv)
    return out.reshape(n, 1)
